# baseline (speedup 1.0000x reference)
import functools

import jax
import jax.numpy as jnp
from jax import lax
from jax.experimental import pallas as pl
from jax.experimental.pallas import tpu as pltpu
from jax.experimental.pallas import tpu_sc as plsc

_BATCH = 16384
_EMB_DIM = 32
_NC = 2
_NS = 16
_NW = _NC * _NS
_B_PER_W = _BATCH // _NW


def kernel(condition, embedding_weight):
    mesh = plsc.VectorSubcoreMesh(core_axis_name="c", subcore_axis_name="s")
    t4 = embedding_weight.reshape(250000, 128)
    idx = condition.astype(jnp.int32)

    @functools.partial(
        pl.kernel,
        mesh=mesh,
        out_type=jax.ShapeDtypeStruct((_BATCH, _EMB_DIM), jnp.float32),
        scratch_types=[
            pltpu.VMEM((_B_PER_W,), jnp.int32),
            pltpu.VMEM((_B_PER_W,), jnp.int32),
            pltpu.VMEM((_B_PER_W, 128), jnp.float32),
            pltpu.VMEM((_B_PER_W, _EMB_DIM), jnp.float32),
            pltpu.SemaphoreType.DMA,
        ],
        compiler_params=pltpu.CompilerParams(use_tc_tiling_on_sc=False),
    )
    def k(table_hbm, idx_hbm, out_hbm, idx_v, q_v, rows_v, out_v, sem):
        wid = lax.axis_index("s") * _NC + lax.axis_index("c")
        base = wid * _B_PER_W
        pltpu.sync_copy(idx_hbm.at[pl.ds(base, _B_PER_W)], idx_v)

        @pl.loop(0, _B_PER_W, step=16)
        def _(i0):
            q_v[pl.ds(i0, 16)] = idx_v[pl.ds(i0, 16)] >> 2

        pltpu.async_copy(table_hbm.at[q_v], rows_v, sem).wait()

        @pl.loop(0, _B_PER_W, step=16)
        def _(j0):
            v = idx_v[pl.ds(j0, 16)]
            for t in range(16):
                off = (v[t] & 3) * _EMB_DIM
                out_v[j0 + t, pl.ds(0, 16)] = rows_v[j0 + t, pl.ds(off, 16)]
                out_v[j0 + t, pl.ds(16, 16)] = rows_v[
                    j0 + t, pl.ds(off + 16, 16)
                ]

        pltpu.sync_copy(out_v, out_hbm.at[pl.ds(base, _B_PER_W)])

    return k(t4, idx)
